# Initial kernel scaffold; baseline (speedup 1.0000x reference)
import jax
import jax.numpy as jnp
from jax.experimental import pallas as pl


def kernel(box_cls, box_regression, anchors):
    raise NotImplementedError("write your pallas kernel here")


# speedup vs baseline: 11.5054x; 11.5054x over previous
import math

import jax
import jax.numpy as jnp
from jax.experimental import pallas as pl
from jax.experimental.pallas import tpu as pltpu

B = 2
N = 20000
PRE = 2000
POST = 1000
NP = 2048
T = 128
NBLK = NP // T
OUT = 1024
IMW = 800.0
IMH = 800.0
MIN_SIZE = 1.0
SCORE_TH = 0.0
NMS_TH = 0.7
BBOX_XFORM_CLIP = math.log(1000.0 / 16.0)


def _rpn_kernel(sc_ref, anc_ref, dlt_ref, ob_ref, os_ref, iou_scr):
    anc = anc_ref[0]
    dlt = dlt_ref[0]
    raw = sc_ref[0]

    x0a = anc[0:1, :]
    y0a = anc[1:2, :]
    x1a = anc[2:3, :]
    y1a = anc[3:4, :]
    dx = dlt[0:1, :]
    dy = dlt[1:2, :]
    dw = jnp.minimum(dlt[2:3, :], BBOX_XFORM_CLIP)
    dh = jnp.minimum(dlt[3:4, :], BBOX_XFORM_CLIP)

    wa = x1a - x0a
    ha = y1a - y0a
    cxa = x0a + 0.5 * wa
    cya = y0a + 0.5 * ha
    pcx = dx * wa + cxa
    pcy = dy * ha + cya
    pw = jnp.exp(dw) * wa
    ph = jnp.exp(dh) * ha

    bx0 = jnp.clip(pcx - 0.5 * pw, 0.0, IMW)
    by0 = jnp.clip(pcy - 0.5 * ph, 0.0, IMH)
    bx1 = jnp.clip(pcx + 0.5 * pw, 0.0, IMW)
    by1 = jnp.clip(pcy + 0.5 * ph, 0.0, IMH)

    ws = bx1 - bx0
    hs = by1 - by0
    areas = ws * hs
    sc = jax.nn.sigmoid(raw)

    valid = (ws >= MIN_SIZE) & (hs >= MIN_SIZE) & (sc >= SCORE_TH)
    K0 = jnp.where(valid, 1.0, 0.0)
    kblocks = [K0[:, j * T:(j + 1) * T] for j in range(NBLK)]

    x0m = bx0.reshape(NBLK, T)
    y0m = by0.reshape(NBLK, T)
    x1m = bx1.reshape(NBLK, T)
    y1m = by1.reshape(NBLK, T)
    am = areas.reshape(NBLK, T)

    lane_t = jax.lax.broadcasted_iota(jnp.int32, (1, T), 1)
    lane_np = jax.lax.broadcasted_iota(jnp.int32, (1, NP), 1)
    row_t = jax.lax.broadcasted_iota(jnp.int32, (T, 1), 0)
    ui = jax.lax.broadcasted_iota(jnp.int32, (T, T), 0)
    uj = jax.lax.broadcasted_iota(jnp.int32, (T, T), 1)
    eye = jnp.where(ui == uj, 1.0, 0.0)

    for bi in range(NBLK):
        s = bi * T
        c0 = x0m[bi][:, None]
        c1 = y0m[bi][:, None]
        c2 = x1m[bi][:, None]
        c3 = y1m[bi][:, None]
        ca = am[bi][:, None]

        ltx = jnp.maximum(c0, x0m[bi][None, :])
        lty = jnp.maximum(c1, y0m[bi][None, :])
        rbx = jnp.minimum(c2, x1m[bi][None, :])
        rby = jnp.minimum(c3, y1m[bi][None, :])
        iw = jnp.maximum(rbx - ltx, 0.0)
        ih = jnp.maximum(rby - lty, 0.0)
        inter = iw * ih
        uni = ca + am[bi][None, :] - inter + 1e-6
        iou_scr[...] = jnp.where(inter > NMS_TH * uni, 1.0, 0.0)

        def body(t, kb):
            kt = jnp.sum(jnp.where(lane_t == t, kb, 0.0))
            row = iou_scr[pl.ds(t, 1), :]
            sup = (row > 0.5) & (lane_t > t) & (kt > 0.0)
            return jnp.where(sup, 0.0, kb)

        kb = jax.lax.fori_loop(0, T, body, kblocks[bi])
        kblocks[bi] = kb

        kbc = jax.lax.dot_general(
            eye, kb, (((1,), (1,)), ((), ())),
            preferred_element_type=jnp.float32)
        gix = s + row_t
        ltx = jnp.maximum(c0, bx0)
        lty = jnp.maximum(c1, by0)
        rbx = jnp.minimum(c2, bx1)
        rby = jnp.minimum(c3, by1)
        iw = jnp.maximum(rbx - ltx, 0.0)
        ih = jnp.maximum(rby - lty, 0.0)
        inter = iw * ih
        uni = ca + areas - inter + 1e-6
        sup = (inter > NMS_TH * uni) & (kbc > 0.0) & (lane_np > gix)
        sup_any = jnp.max(jnp.where(sup, 1.0, 0.0), axis=0, keepdims=True)
        for j in range(bi + 1, NBLK):
            kblocks[j] = jnp.where(
                sup_any[:, j * T:(j + 1) * T] > 0.0, 0.0, kblocks[j])

    K = jnp.concatenate(kblocks, axis=1)
    Km = K.reshape(NBLK, T)
    U = jnp.where(ui <= uj, 1.0, 0.0)
    intra = jnp.dot(Km, U, preferred_element_type=jnp.float32)
    tot = jnp.sum(Km, axis=1, keepdims=True)
    mr = jax.lax.broadcasted_iota(jnp.int32, (NBLK, NBLK), 0)
    mq = jax.lax.broadcasted_iota(jnp.int32, (NBLK, NBLK), 1)
    M2 = jnp.where(mq < mr, 1.0, 0.0)
    off = jnp.dot(M2, tot, preferred_element_type=jnp.float32)
    p = intra + off - 1.0

    prow = p.astype(jnp.int32).reshape(1, NP)
    jcol = jax.lax.broadcasted_iota(jnp.int32, (OUT, 1), 0)
    P = jnp.where((jcol == prow) & (K > 0.0), 1.0, 0.0)

    data = jnp.concatenate([bx0, by0, bx1, by1, sc], axis=0)
    out = jax.lax.dot_general(
        data, P, (((1,), (1,)), ((), ())),
        preferred_element_type=jnp.float32)

    ob_ref[0] = out[0:4, :]
    os_ref[0] = out[4:5, :]


def kernel(box_cls, box_regression, anchors):
    scores = box_cls.reshape(B, N)
    top_scores, top_idx = jax.lax.top_k(scores, PRE)
    ga = jnp.take_along_axis(
        anchors.reshape(B, N, 4), top_idx[..., None], axis=1)
    gd = jnp.take_along_axis(
        box_regression.reshape(B, N, 4), top_idx[..., None], axis=1)

    pad = NP - PRE
    sc_p = jnp.pad(top_scores, ((0, 0), (0, pad))).reshape(B, 1, NP)
    ga_t = jnp.pad(ga.transpose(0, 2, 1), ((0, 0), (0, 0), (0, pad)))
    gd_t = jnp.pad(gd.transpose(0, 2, 1), ((0, 0), (0, 0), (0, pad)))

    outb, outs = pl.pallas_call(
        _rpn_kernel,
        grid=(B,),
        in_specs=[
            pl.BlockSpec((1, 1, NP), lambda b: (b, 0, 0)),
            pl.BlockSpec((1, 4, NP), lambda b: (b, 0, 0)),
            pl.BlockSpec((1, 4, NP), lambda b: (b, 0, 0)),
        ],
        out_specs=[
            pl.BlockSpec((1, 4, OUT), lambda b: (b, 0, 0)),
            pl.BlockSpec((1, 1, OUT), lambda b: (b, 0, 0)),
        ],
        out_shape=[
            jax.ShapeDtypeStruct((B, 4, OUT), jnp.float32),
            jax.ShapeDtypeStruct((B, 1, OUT), jnp.float32),
        ],
        scratch_shapes=[pltpu.VMEM((T, T), jnp.float32)],
    )(sc_p, ga_t, gd_t)

    boxes = outb[:, :, :POST].transpose(0, 2, 1)
    return boxes, outs.reshape(B, OUT)[:, :POST]


# speedup vs baseline: 35.4769x; 3.0835x over previous
import math

import jax
import jax.numpy as jnp
from jax.experimental import pallas as pl
from jax.experimental.pallas import tpu as pltpu

B = 2
N = 20000
PRE = 2000
POST = 1000
NP = 2048
T = 128
NBLK = NP // T
OUT = 1024
IMW = 800.0
IMH = 800.0
MIN_SIZE = 1.0
SCORE_TH = 0.0
NMS_TH = 0.7
BBOX_XFORM_CLIP = math.log(1000.0 / 16.0)


def _rpn_kernel(sc_ref, anc_ref, dlt_ref, ob_ref, os_ref, iou_scr):
    anc = anc_ref[0]
    dlt = dlt_ref[0]
    raw = sc_ref[0]

    x0a = anc[0:1, :]
    y0a = anc[1:2, :]
    x1a = anc[2:3, :]
    y1a = anc[3:4, :]
    dx = dlt[0:1, :]
    dy = dlt[1:2, :]
    dw = jnp.minimum(dlt[2:3, :], BBOX_XFORM_CLIP)
    dh = jnp.minimum(dlt[3:4, :], BBOX_XFORM_CLIP)

    wa = x1a - x0a
    ha = y1a - y0a
    cxa = x0a + 0.5 * wa
    cya = y0a + 0.5 * ha
    pcx = dx * wa + cxa
    pcy = dy * ha + cya
    pw = jnp.exp(dw) * wa
    ph = jnp.exp(dh) * ha

    bx0 = jnp.clip(pcx - 0.5 * pw, 0.0, IMW)
    by0 = jnp.clip(pcy - 0.5 * ph, 0.0, IMH)
    bx1 = jnp.clip(pcx + 0.5 * pw, 0.0, IMW)
    by1 = jnp.clip(pcy + 0.5 * ph, 0.0, IMH)

    ws = bx1 - bx0
    hs = by1 - by0
    areas = ws * hs
    sc = jax.nn.sigmoid(raw)

    valid = (ws >= MIN_SIZE) & (hs >= MIN_SIZE) & (sc >= SCORE_TH)
    K0 = jnp.where(valid, 1.0, 0.0)
    kblocks = [K0[:, j * T:(j + 1) * T] for j in range(NBLK)]

    x0m = bx0.reshape(NBLK, T)
    y0m = by0.reshape(NBLK, T)
    x1m = bx1.reshape(NBLK, T)
    y1m = by1.reshape(NBLK, T)
    am = areas.reshape(NBLK, T)

    lane_t = jax.lax.broadcasted_iota(jnp.int32, (1, T), 1)
    lane_np = jax.lax.broadcasted_iota(jnp.int32, (1, NP), 1)
    row_t = jax.lax.broadcasted_iota(jnp.int32, (T, 1), 0)
    ui = jax.lax.broadcasted_iota(jnp.int32, (T, T), 0)
    uj = jax.lax.broadcasted_iota(jnp.int32, (T, T), 1)
    eye = jnp.where(ui == uj, 1.0, 0.0)

    for bi in range(NBLK):
        s = bi * T
        c0 = x0m[bi][:, None]
        c1 = y0m[bi][:, None]
        c2 = x1m[bi][:, None]
        c3 = y1m[bi][:, None]
        ca = am[bi][:, None]

        ltx = jnp.maximum(c0, x0m[bi][None, :])
        lty = jnp.maximum(c1, y0m[bi][None, :])
        rbx = jnp.minimum(c2, x1m[bi][None, :])
        rby = jnp.minimum(c3, y1m[bi][None, :])
        iw = jnp.maximum(rbx - ltx, 0.0)
        ih = jnp.maximum(rby - lty, 0.0)
        inter = iw * ih
        uni = ca + am[bi][None, :] - inter + 1e-6
        iou_scr[...] = jnp.where(
            (inter > NMS_TH * uni) & (ui < uj), 1.0, 0.0)

        kb_init = kblocks[bi]

        def cond_fn(st):
            return st[1] > 0.0

        def body_fn(st):
            kb, _, it = st
            sup = jnp.dot(kb, iou_scr[...],
                          preferred_element_type=jnp.float32)
            kb_new = jnp.where(sup < 0.5, kb_init, 0.0)
            changed = jnp.sum(jnp.abs(kb_new - kb))
            changed = jnp.where(it < T, changed, 0.0)
            return kb_new, changed, it + 1

        kb, _, _ = jax.lax.while_loop(
            cond_fn, body_fn,
            (kb_init, jnp.float32(1.0), jnp.int32(0)))
        kblocks[bi] = kb

        kbc = jax.lax.dot_general(
            eye, kb, (((1,), (1,)), ((), ())),
            preferred_element_type=jnp.float32)
        gix = s + row_t
        ltx = jnp.maximum(c0, bx0)
        lty = jnp.maximum(c1, by0)
        rbx = jnp.minimum(c2, bx1)
        rby = jnp.minimum(c3, by1)
        iw = jnp.maximum(rbx - ltx, 0.0)
        ih = jnp.maximum(rby - lty, 0.0)
        inter = iw * ih
        uni = ca + areas - inter + 1e-6
        sup = (inter > NMS_TH * uni) & (kbc > 0.0) & (lane_np > gix)
        sup_any = jnp.max(jnp.where(sup, 1.0, 0.0), axis=0, keepdims=True)
        for j in range(bi + 1, NBLK):
            kblocks[j] = jnp.where(
                sup_any[:, j * T:(j + 1) * T] > 0.0, 0.0, kblocks[j])

    K = jnp.concatenate(kblocks, axis=1)
    Km = K.reshape(NBLK, T)
    U = jnp.where(ui <= uj, 1.0, 0.0)
    intra = jnp.dot(Km, U, preferred_element_type=jnp.float32)
    tot = jnp.sum(Km, axis=1, keepdims=True)
    mr = jax.lax.broadcasted_iota(jnp.int32, (NBLK, NBLK), 0)
    mq = jax.lax.broadcasted_iota(jnp.int32, (NBLK, NBLK), 1)
    M2 = jnp.where(mq < mr, 1.0, 0.0)
    off = jnp.dot(M2, tot, preferred_element_type=jnp.float32)
    p = intra + off - 1.0

    prow = p.astype(jnp.int32).reshape(1, NP)
    jcol = jax.lax.broadcasted_iota(jnp.int32, (OUT, 1), 0)
    P = jnp.where((jcol == prow) & (K > 0.0), 1.0, 0.0)

    data = jnp.concatenate([bx0, by0, bx1, by1, sc], axis=0)
    out = jax.lax.dot_general(
        data, P, (((1,), (1,)), ((), ())),
        preferred_element_type=jnp.float32)

    ob_ref[0] = out[0:4, :]
    os_ref[0] = out[4:5, :]


def kernel(box_cls, box_regression, anchors):
    scores = box_cls.reshape(B, N)
    top_scores, top_idx = jax.lax.top_k(scores, PRE)
    ga = jnp.take_along_axis(
        anchors.reshape(B, N, 4), top_idx[..., None], axis=1)
    gd = jnp.take_along_axis(
        box_regression.reshape(B, N, 4), top_idx[..., None], axis=1)

    pad = NP - PRE
    sc_p = jnp.pad(top_scores, ((0, 0), (0, pad))).reshape(B, 1, NP)
    ga_t = jnp.pad(ga.transpose(0, 2, 1), ((0, 0), (0, 0), (0, pad)))
    gd_t = jnp.pad(gd.transpose(0, 2, 1), ((0, 0), (0, 0), (0, pad)))

    outb, outs = pl.pallas_call(
        _rpn_kernel,
        grid=(B,),
        in_specs=[
            pl.BlockSpec((1, 1, NP), lambda b: (b, 0, 0)),
            pl.BlockSpec((1, 4, NP), lambda b: (b, 0, 0)),
            pl.BlockSpec((1, 4, NP), lambda b: (b, 0, 0)),
        ],
        out_specs=[
            pl.BlockSpec((1, 4, OUT), lambda b: (b, 0, 0)),
            pl.BlockSpec((1, 1, OUT), lambda b: (b, 0, 0)),
        ],
        out_shape=[
            jax.ShapeDtypeStruct((B, 4, OUT), jnp.float32),
            jax.ShapeDtypeStruct((B, 1, OUT), jnp.float32),
        ],
        scratch_shapes=[pltpu.VMEM((T, T), jnp.float32)],
    )(sc_p, ga_t, gd_t)

    boxes = outb[:, :, :POST].transpose(0, 2, 1)
    return boxes, outs.reshape(B, OUT)[:, :POST]
